# baseline (speedup 1.0000x reference)
import functools

import jax
import jax.numpy as jnp
from jax import lax
from jax.experimental import pallas as pl
from jax.experimental.pallas import tpu as pltpu
from jax.experimental.pallas import tpu_sc as plsc

_C = 5
_T = 512
_ROWS = 10240
_NC = 2
_NS = 16
_NW = _NC * _NS
_L = 16
_RPW = _ROWS // _NW
_CH = 2
_GPW = _RPW // (_L * _CH)
_UNROLL = 4


def _sc_call(vflat, tgt):
    mesh = plsc.VectorSubcoreMesh(core_axis_name="c", subcore_axis_name="s")

    @functools.partial(
        pl.kernel, mesh=mesh,
        compiler_params=pltpu.CompilerParams(needs_layout_passes=False),
        out_type=[
            jax.ShapeDtypeStruct((_ROWS,), jnp.float32),
            jax.ShapeDtypeStruct((_NW * _L,), jnp.float32),
        ],
        scratch_types=[
            pltpu.VMEM((_CH * _L * _T,), jnp.float32),
            pltpu.VMEM((_CH * _L * _T,), jnp.float32),
            pltpu.VMEM((_RPW,), jnp.float32),
            pltpu.VMEM((_RPW,), jnp.float32),
            pltpu.VMEM((_L,), jnp.float32),
            pltpu.SemaphoreType.DMA,
            pltpu.SemaphoreType.DMA,
        ],
    )
    def _stca_sc(v_hbm, tgt_hbm, spike_hbm, lpart_hbm,
                 vbuf_a, vbuf_b, tgt_buf, spike_buf, loss_buf, sem_a, sem_b):
        wid = lax.axis_index("s") * _NC + lax.axis_index("c")
        base_row = wid * _RPW
        pltpu.sync_copy(tgt_hbm.at[pl.ds(base_row, _RPW)], tgt_buf)

        bufs = (vbuf_a, vbuf_b)
        sems = (sem_a, sem_b)
        pair_elems = _CH * _L * _T

        def fetch(g):
            return pltpu.async_copy(
                v_hbm.at[pl.ds(base_row * _T + g * pair_elems, pair_elems)],
                bufs[g % 2], sems[g % 2])

        lanes = lax.iota(jnp.int32, _L)
        zero = jnp.zeros((_L,), jnp.float32)
        one = jnp.full((_L,), 1.0, jnp.float32)
        five = jnp.full((_L,), float(_C), jnp.float32)
        big = jnp.full((_L,), 1e30, jnp.float32)
        half = jnp.full((_L,), 0.5, jnp.float32)
        neg = jnp.full((_L,), -1e30, jnp.float32)
        base_idx = lanes * _T
        loss_acc = zero

        def one_step(vbuf, s):
            (idx, since, cnt, psum, pn, bc, bps, bpn, ncl, vmax) = s
            v = plsc.load_gather(vbuf, [idx])
            pos = v >= zero
            poss = v > zero
            st = pos & (since > five)
            close = st & (cnt < bc)
            bc = jnp.where(close, cnt, bc)
            bps = jnp.where(close, psum, bps)
            bpn = jnp.where(close, pn, bpn)
            inc_c = jnp.where(pos, one, zero)
            sv = jnp.where(poss, v, zero)
            inc_s = jnp.where(poss, one, zero)
            cnt = jnp.where(st, one, cnt + inc_c)
            psum = jnp.where(st, sv, psum + sv)
            pn = jnp.where(st, inc_s, pn + inc_s)
            ncl = ncl + jnp.where(st, one, zero)
            vmax = jnp.maximum(vmax, v)
            since = jnp.where(pos, one, since + one)
            return (idx + 1, since, cnt, psum, pn, bc, bps, bpn, ncl, vmax)

        def finish(s, goff):
            (_, _, cnt, psum, pn, bc, bps, bpn, ncl, vmax) = s
            close = cnt < bc
            bps = jnp.where(close, psum, bps)
            bpn = jnp.where(close, pn, bpn)
            tgtv = plsc.load_gather(tgt_buf, [goff])
            is_tgt = tgtv > half
            spiked = ncl > half
            contrib = jnp.where(bpn > zero, bps / jnp.maximum(bpn, one), zero)
            rowloss = jnp.where(is_tgt & ~spiked, -vmax,
                                jnp.where((~is_tgt) & spiked, contrib, zero))
            plsc.store_scatter(spike_buf, [goff], ncl)
            return rowloss

        pending = fetch(0)
        for g in range(_GPW):
            pending.wait()
            if g + 1 < _GPW:
                pending = fetch(g + 1)
            vbuf = bufs[g % 2]

            def step(_, carry, vbuf=vbuf):
                s0, s1 = carry
                for _u in range(_UNROLL):
                    s0 = one_step(vbuf, s0)
                    s1 = one_step(vbuf, s1)
                return (s0, s1)

            def init(chain):
                return (base_idx + chain * (_L * _T), big, big, zero, zero,
                        big, zero, zero, zero, neg)

            s0, s1 = lax.fori_loop(0, _T // _UNROLL, step, (init(0), init(1)))
            loss_acc = loss_acc + finish(s0, lanes + (2 * g) * _L)
            loss_acc = loss_acc + finish(s1, lanes + (2 * g + 1) * _L)

        loss_buf[...] = loss_acc
        pltpu.sync_copy(spike_buf, spike_hbm.at[pl.ds(base_row, _RPW)])
        pltpu.sync_copy(loss_buf, lpart_hbm.at[pl.ds(wid * _L, _L)])

    return _stca_sc(vflat, tgt)


@jax.jit
def _run(vmem, labels):
    B, N, T = vmem.shape
    tgt = (labels[:, None] == jnp.arange(N, dtype=labels.dtype)[None, :])
    spike, lpart = _sc_call(vmem.reshape(-1), tgt.reshape(-1).astype(jnp.float32))
    return jnp.sum(lpart), spike.reshape(B, N)


def kernel(vmem, vlastmem, labels):
    del vlastmem
    return _run(vmem, labels)
